# baseline (speedup 1.0000x reference)
import jax
import jax.numpy as jnp
from jax.experimental import pallas as pl
from jax.experimental.pallas import tpu as pltpu

TOPK = 2
NEXP = 8
BLOCK = 1024


def _gate_kernel(x_ref, w_ref, idx_ref, wgt_ref):
    x = x_ref[...]
    w = w_ref[...]
    logits = jax.lax.dot_general(
        w, x, (((1,), (1,)), ((), ())), preferred_element_type=jnp.float32
    )
    T = logits.shape[1]
    iota = jax.lax.broadcasted_iota(jnp.int32, (NEXP, T), 0)

    m1 = jnp.max(logits, axis=0, keepdims=True)
    idx1 = jnp.min(jnp.where(logits == m1, iota, NEXP), axis=0, keepdims=True)

    masked = jnp.where(iota == idx1, -jnp.inf, logits)
    m2 = jnp.max(masked, axis=0, keepdims=True)
    idx2 = jnp.min(jnp.where(masked == m2, iota, NEXP), axis=0, keepdims=True)

    z = jnp.sum(jnp.exp(logits - m1), axis=0, keepdims=True)
    w1 = 1.0 / z
    w2 = jnp.exp(m2 - m1) / z

    idx_ref[...] = jnp.concatenate([idx1, idx2], axis=0)
    wgt_ref[...] = jnp.concatenate([w1, w2], axis=0)


@jax.jit
def kernel(hidden_states, weight):
    bsz, seq_len, h = hidden_states.shape
    n = bsz * seq_len
    x = hidden_states.reshape(n, h)

    grid = (n // BLOCK,)
    idx_t, wgt_t = pl.pallas_call(
        _gate_kernel,
        grid=grid,
        in_specs=[
            pl.BlockSpec((BLOCK, h), lambda i: (i, 0)),
            pl.BlockSpec((NEXP, h), lambda i: (0, 0)),
        ],
        out_specs=[
            pl.BlockSpec((TOPK, BLOCK), lambda i: (0, i)),
            pl.BlockSpec((TOPK, BLOCK), lambda i: (0, i)),
        ],
        out_shape=[
            jax.ShapeDtypeStruct((TOPK, n), jnp.int32),
            jax.ShapeDtypeStruct((TOPK, n), jnp.float32),
        ],
        compiler_params=pltpu.CompilerParams(
            dimension_semantics=("arbitrary",),
            allow_input_fusion=[True, True],
        ),
    )(x, weight)
    return idx_t.T, wgt_t.T
